# baseline (speedup 1.0000x reference)
import functools

import jax
import jax.numpy as jnp
from jax import lax
from jax.experimental import pallas as pl
from jax.experimental.pallas import tpu as pltpu
from jax.experimental.pallas import tpu_sc as plsc

N = 10000
D = 64
W128 = 128
E = 320000
NC = 2
NS = 16
NW = NC * NS
CS = 128
CH0 = 55
CH1 = 104
CH = CH1
EPAD0 = NS * CH0 * CS
EPAD1 = NS * CH1 * CS
R = 10112
RT = R // NS

_mesh = plsc.VectorSubcoreMesh(core_axis_name="c", subcore_axis_name="s",
                               num_cores=NC, num_subcores=NS)


@functools.partial(
    pl.kernel,
    out_type=jax.ShapeDtypeStruct((NC, R, W128), jnp.float32),
    mesh=_mesh,
    scratch_types=[
        pltpu.VMEM((CH, CS), jnp.int32),
        pltpu.VMEM((CH, CS), jnp.int32),
        pltpu.VMEM((CS, W128), jnp.float32),
        pltpu.VMEM_SHARED((R, W128), jnp.float32),
        pltpu.SemaphoreType.DMA,
    ],
)
def _sc_agg(table_hbm, src_hbm, dst_hbm, z_hbm, out_hbm,
            src_v, dst_v, rows_v, acc, semg):
    c = lax.axis_index("c")
    s = lax.axis_index("s")
    wid = c * NS + s
    pltpu.sync_copy(z_hbm.at[pl.ds(s * RT, RT)], acc.at[pl.ds(s * RT, RT)])
    pltpu.sync_copy(src_hbm.at[wid], src_v)
    pltpu.sync_copy(dst_hbm.at[wid], dst_v)
    plsc.subcore_barrier()

    def body(j, carry):
        pltpu.async_copy(table_hbm.at[src_v.at[j]], rows_v, semg).wait()
        pltpu.sync_copy(rows_v, acc.at[dst_v.at[j]], add=True)
        return carry

    nch = jnp.where(c == 0, CH0, CH1)
    lax.fori_loop(0, nch, body, 0)
    plsc.subcore_barrier()
    pltpu.sync_copy(acc.at[pl.ds(s * RT, RT)],
                    out_hbm.at[c, pl.ds(s * RT, RT)])


def _lrelu(v):
    return jnp.where(v >= 0, v, 0.01 * v)


def _mlp_body(f_ref, w_ref, b_ref, o_ref):
    o_ref[...] = (jnp.dot(f_ref[...], w_ref[...],
                          preferred_element_type=jnp.float32) + b_ref[...])


_BR = RT


def _table(xn):
    br = xn.shape[0]
    return jnp.concatenate(
        [xn, jnp.ones((br, 1), jnp.float32),
         jnp.zeros((br, W128 - D - 1), jnp.float32)], axis=1)


def _norm_body(x_ref, o_ref):
    x = x_ref[...]
    n = jnp.sqrt(jnp.sum(x * x, axis=1, keepdims=True))
    o_ref[...] = _table(x / jnp.maximum(n, 1e-12))


def _layer_body(xt_ref, p_ref, c_ref, id_ref,
                cw_ref, lw_ref, lb_ref, gw_ref, gb_ref, o_ref):
    cnt = jnp.maximum(c_ref[0] + c_ref[1], 1.0)[:, 0:1]
    sagg = (p_ref[0, :, :D] + p_ref[1, :, :D]) / cnt
    h = _lrelu(jnp.dot(sagg, cw_ref[...], preferred_element_type=jnp.float32))
    x_hat = _lrelu(jnp.dot(xt_ref[:, :D], lw_ref[...],
                           preferred_element_type=jnp.float32)
                   + lb_ref[...]) + id_ref[...]
    o_ref[...] = _table(_lrelu(jnp.dot(h, gw_ref[...],
                                       preferred_element_type=jnp.float32)
                               + gb_ref[...] + x_hat))


def _tc_mlp(features, w, b):
    m = features.shape[0]
    return pl.pallas_call(
        _mlp_body,
        grid=(m // 1000,),
        in_specs=[
            pl.BlockSpec((1000, 128), lambda i: (i, 0)),
            pl.BlockSpec((128, D), lambda i: (0, 0)),
            pl.BlockSpec((1, D), lambda i: (0, 0)),
        ],
        out_specs=pl.BlockSpec((1000, D), lambda i: (i, 0)),
        out_shape=jax.ShapeDtypeStruct((m, D), jnp.float32),
    )(features, w, b.reshape(1, D))


def _tc_norm(xpad):
    return pl.pallas_call(
        _norm_body,
        grid=(R // _BR,),
        in_specs=[pl.BlockSpec((_BR, D), lambda i: (i, 0))],
        out_specs=pl.BlockSpec((_BR, W128), lambda i: (i, 0)),
        out_shape=jax.ShapeDtypeStruct((R, W128), jnp.float32),
    )(xpad)


def _tc_layer(xt, p, cnt_src, id_pad, cw, lw, lb, gw, gb):
    mat = pl.BlockSpec((D, D), lambda i: (0, 0))
    vec = pl.BlockSpec((1, D), lambda i: (0, 0))
    return pl.pallas_call(
        _layer_body,
        grid=(R // _BR,),
        in_specs=[pl.BlockSpec((_BR, W128), lambda i: (i, 0)),
                  pl.BlockSpec((NC, _BR, W128), lambda i: (0, i, 0)),
                  pl.BlockSpec((NC, _BR, 16), lambda i: (0, i, 0)),
                  pl.BlockSpec((_BR, D), lambda i: (i, 0)),
                  mat, mat, vec, mat, vec],
        out_specs=pl.BlockSpec((_BR, W128), lambda i: (i, 0)),
        out_shape=jax.ShapeDtypeStruct((R, W128), jnp.float32),
    )(xt, p, cnt_src, id_pad, cw, lw, lb.reshape(1, D), gw,
      gb.reshape(1, D))


def kernel(features, id_embedding, edge_index, preference, W_mlp, b_mlp,
           conv1_w, lin1_w, lin1_b, g1_w, g1_b,
           conv2_w, lin2_w, lin2_b, g2_w, g2_b,
           conv3_w, lin3_w, lin3_b, g3_w, g3_b):
    src = edge_index[0]
    dst = edge_index[1]
    pad = EPAD0 + EPAD1 - E
    paddst = N + (jnp.arange(pad, dtype=jnp.int32) % (R - N))
    chpad = jnp.zeros((NS, CH - CH0, CS), jnp.int32)

    def _split(idx, padv):
        a = idx[:EPAD0].reshape(NS, CH0, CS)
        a = jnp.concatenate([a, chpad], axis=1)
        b = jnp.concatenate([idx[EPAD0:], padv]).reshape(NS, CH1, CS)
        return jnp.concatenate([a, b], axis=0)

    srcp = _split(src, jnp.zeros((pad,), jnp.int32))
    dstp = _split(dst, paddst)
    zeros128 = jnp.zeros((R, W128), jnp.float32)
    padrows = jnp.zeros((R - N, D), jnp.float32)
    id_pad = jnp.concatenate([id_embedding, padrows])

    temp = _tc_mlp(features, W_mlp, b_mlp)
    xpad = jnp.concatenate([preference, temp, padrows[:R - N]], axis=0)
    xt = _tc_norm(xpad)

    cnt_src = None
    for cw, lw, lb, gw, gb in (
        (conv1_w, lin1_w, lin1_b, g1_w, g1_b),
        (conv2_w, lin2_w, lin2_b, g2_w, g2_b),
        (conv3_w, lin3_w, lin3_b, g3_w, g3_b),
    ):
        p = _sc_agg(xt, srcp, dstp, zeros128)
        if cnt_src is None:
            cnt_src = p[:, :, D:D + 16]
        xt = _tc_layer(xt, p, cnt_src, id_pad, cw, lw, lb, gw, gb)
    return xt[:N, :D]


# speedup vs baseline: 1.1278x; 1.1278x over previous
import functools

import jax
import jax.numpy as jnp
from jax import lax
from jax.experimental import pallas as pl
from jax.experimental.pallas import tpu as pltpu
from jax.experimental.pallas import tpu_sc as plsc

N = 10000
D = 64
W128 = 128
E = 320000
NC = 2
NS = 16
NW = NC * NS
CS = 128
CH0 = 104
CH1 = 55
CH = max(CH0, CH1)
EPAD0 = NS * CH0 * CS
EPAD1 = NS * CH1 * CS
R = 10112
RT = R // NS

_mesh = plsc.VectorSubcoreMesh(core_axis_name="c", subcore_axis_name="s",
                               num_cores=NC, num_subcores=NS)


@functools.partial(
    pl.kernel,
    out_type=jax.ShapeDtypeStruct((NC, R, W128), jnp.float32),
    mesh=_mesh,
    scratch_types=[
        pltpu.VMEM((CH, CS), jnp.int32),
        pltpu.VMEM((CH, CS), jnp.int32),
        pltpu.VMEM((CS, W128), jnp.float32),
        pltpu.VMEM_SHARED((R, W128), jnp.float32),
        pltpu.SemaphoreType.DMA,
    ],
)
def _sc_agg(table_hbm, src_hbm, dst_hbm, z_hbm, out_hbm,
            src_v, dst_v, rows_v, acc, semg):
    c = lax.axis_index("c")
    s = lax.axis_index("s")
    wid = c * NS + s
    pltpu.sync_copy(z_hbm.at[pl.ds(s * RT, RT)], acc.at[pl.ds(s * RT, RT)])
    pltpu.sync_copy(src_hbm.at[wid], src_v)
    pltpu.sync_copy(dst_hbm.at[wid], dst_v)
    plsc.subcore_barrier()

    def body(j, carry):
        pltpu.async_copy(table_hbm.at[src_v.at[j]], rows_v, semg).wait()
        pltpu.sync_copy(rows_v, acc.at[dst_v.at[j]], add=True)
        return carry

    nch = jnp.where(c == 0, CH0, CH1)
    lax.fori_loop(0, nch, body, 0)
    plsc.subcore_barrier()
    pltpu.sync_copy(acc.at[pl.ds(s * RT, RT)],
                    out_hbm.at[c, pl.ds(s * RT, RT)])


def _lrelu(v):
    return jnp.where(v >= 0, v, 0.01 * v)


def _mlp_body(f_ref, w_ref, b_ref, o_ref):
    o_ref[...] = (jnp.dot(f_ref[...], w_ref[...],
                          preferred_element_type=jnp.float32) + b_ref[...])


_BR = RT


def _table(xn):
    br = xn.shape[0]
    return jnp.concatenate(
        [xn, jnp.ones((br, 1), jnp.float32),
         jnp.zeros((br, W128 - D - 1), jnp.float32)], axis=1)


def _norm_body(x_ref, o_ref):
    x = x_ref[...]
    n = jnp.sqrt(jnp.sum(x * x, axis=1, keepdims=True))
    o_ref[...] = _table(x / jnp.maximum(n, 1e-12))


def _layer_body(xt_ref, p_ref, c_ref, id_ref,
                cw_ref, lw_ref, lb_ref, gw_ref, gb_ref, o_ref):
    cnt = jnp.maximum(c_ref[0] + c_ref[1], 1.0)[:, 0:1]
    sagg = (p_ref[0, :, :D] + p_ref[1, :, :D]) / cnt
    h = _lrelu(jnp.dot(sagg, cw_ref[...], preferred_element_type=jnp.float32))
    x_hat = _lrelu(jnp.dot(xt_ref[:, :D], lw_ref[...],
                           preferred_element_type=jnp.float32)
                   + lb_ref[...]) + id_ref[...]
    o_ref[...] = _table(_lrelu(jnp.dot(h, gw_ref[...],
                                       preferred_element_type=jnp.float32)
                               + gb_ref[...] + x_hat))


def _tc_mlp(features, w, b):
    m = features.shape[0]
    return pl.pallas_call(
        _mlp_body,
        grid=(m // 1000,),
        in_specs=[
            pl.BlockSpec((1000, 128), lambda i: (i, 0)),
            pl.BlockSpec((128, D), lambda i: (0, 0)),
            pl.BlockSpec((1, D), lambda i: (0, 0)),
        ],
        out_specs=pl.BlockSpec((1000, D), lambda i: (i, 0)),
        out_shape=jax.ShapeDtypeStruct((m, D), jnp.float32),
    )(features, w, b.reshape(1, D))


def _tc_norm(xpad):
    return pl.pallas_call(
        _norm_body,
        grid=(R // _BR,),
        in_specs=[pl.BlockSpec((_BR, D), lambda i: (i, 0))],
        out_specs=pl.BlockSpec((_BR, W128), lambda i: (i, 0)),
        out_shape=jax.ShapeDtypeStruct((R, W128), jnp.float32),
    )(xpad)


def _tc_layer(xt, p, cnt_src, id_pad, cw, lw, lb, gw, gb):
    mat = pl.BlockSpec((D, D), lambda i: (0, 0))
    vec = pl.BlockSpec((1, D), lambda i: (0, 0))
    return pl.pallas_call(
        _layer_body,
        grid=(R // _BR,),
        in_specs=[pl.BlockSpec((_BR, W128), lambda i: (i, 0)),
                  pl.BlockSpec((NC, _BR, W128), lambda i: (0, i, 0)),
                  pl.BlockSpec((NC, _BR, 16), lambda i: (0, i, 0)),
                  pl.BlockSpec((_BR, D), lambda i: (i, 0)),
                  mat, mat, vec, mat, vec],
        out_specs=pl.BlockSpec((_BR, W128), lambda i: (i, 0)),
        out_shape=jax.ShapeDtypeStruct((R, W128), jnp.float32),
    )(xt, p, cnt_src, id_pad, cw, lw, lb.reshape(1, D), gw,
      gb.reshape(1, D))


def kernel(features, id_embedding, edge_index, preference, W_mlp, b_mlp,
           conv1_w, lin1_w, lin1_b, g1_w, g1_b,
           conv2_w, lin2_w, lin2_b, g2_w, g2_b,
           conv3_w, lin3_w, lin3_b, g3_w, g3_b):
    src = edge_index[0]
    dst = edge_index[1]
    pad = EPAD0 + EPAD1 - E
    paddst = N + (jnp.arange(pad, dtype=jnp.int32) % (R - N))

    def _chpad(blk, ch):
        return jnp.concatenate(
            [blk, jnp.zeros((NS, CH - ch, CS), jnp.int32)], axis=1)

    def _split(idx, padv):
        a = _chpad(idx[:EPAD0].reshape(NS, CH0, CS), CH0)
        b = _chpad(jnp.concatenate([idx[EPAD0:], padv]
                                   ).reshape(NS, CH1, CS), CH1)
        return jnp.concatenate([a, b], axis=0)

    srcp = _split(src, jnp.zeros((pad,), jnp.int32))
    dstp = _split(dst, paddst)
    zeros128 = jnp.zeros((R, W128), jnp.float32)
    padrows = jnp.zeros((R - N, D), jnp.float32)
    id_pad = jnp.concatenate([id_embedding, padrows])

    temp = _tc_mlp(features, W_mlp, b_mlp)
    xpad = jnp.concatenate([preference, temp, padrows[:R - N]], axis=0)
    xt = _tc_norm(xpad)

    cnt_src = None
    for cw, lw, lb, gw, gb in (
        (conv1_w, lin1_w, lin1_b, g1_w, g1_b),
        (conv2_w, lin2_w, lin2_b, g2_w, g2_b),
        (conv3_w, lin3_w, lin3_b, g3_w, g3_b),
    ):
        p = _sc_agg(xt, srcp, dstp, zeros128)
        if cnt_src is None:
            cnt_src = p[:, :, D:D + 16]
        xt = _tc_layer(xt, p, cnt_src, id_pad, cw, lw, lb, gw, gb)
    return xt[:N, :D]


# speedup vs baseline: 1.3567x; 1.2030x over previous
import functools

import jax
import jax.numpy as jnp
from jax import lax
from jax.experimental import pallas as pl
from jax.experimental.pallas import tpu as pltpu
from jax.experimental.pallas import tpu_sc as plsc

N = 10000
D = 64
W128 = 128
E = 320000
NC = 2
NS = 16
NW = NC * NS
CS = 128
CH = 79
EPAD = NW * CH * CS
R = 10112
RT = R // NS

_mesh = plsc.VectorSubcoreMesh(core_axis_name="c", subcore_axis_name="s",
                               num_cores=NC, num_subcores=NS)


@functools.partial(
    pl.kernel,
    out_type=jax.ShapeDtypeStruct((NC, R, W128), jnp.float32),
    mesh=_mesh,
    scratch_types=[
        pltpu.VMEM((CH, CS), jnp.int32),
        pltpu.VMEM((CH, CS), jnp.int32),
        pltpu.VMEM((CS, W128), jnp.float32),
        pltpu.VMEM_SHARED((R, W128), jnp.float32),
        pltpu.SemaphoreType.DMA,
    ],
)
def _sc_agg(table_hbm, src_hbm, dst_hbm, out_hbm,
            src_v, dst_v, rows_v, acc, semg):
    c = lax.axis_index("c")
    s = lax.axis_index("s")
    wid = c * NS + s
    z16 = jnp.zeros((16,), jnp.float32)

    def zrow(r, carry):
        for q in range(W128 // 16):
            rows_v[r, pl.ds(16 * q, 16)] = z16
        return carry

    lax.fori_loop(0, CS, zrow, 0)
    for k in range(RT // CS):
        pltpu.sync_copy(rows_v, acc.at[pl.ds(s * RT + CS * k, CS)])
    pltpu.sync_copy(rows_v.at[pl.ds(0, RT % CS)],
                    acc.at[pl.ds(s * RT + CS * (RT // CS), RT % CS)])
    pltpu.sync_copy(src_hbm.at[wid], src_v)
    pltpu.sync_copy(dst_hbm.at[wid], dst_v)
    plsc.subcore_barrier()

    def body(j, carry):
        pltpu.async_copy(table_hbm.at[src_v.at[j]], rows_v, semg).wait()
        pltpu.sync_copy(rows_v, acc.at[dst_v.at[j]], add=True)
        return carry

    lax.fori_loop(0, CH, body, 0)
    plsc.subcore_barrier()
    pltpu.sync_copy(acc.at[pl.ds(s * RT, RT)],
                    out_hbm.at[c, pl.ds(s * RT, RT)])


def _lrelu(v):
    return jnp.where(v >= 0, v, 0.01 * v)


def _mlp_body(f_ref, w_ref, b_ref, o_ref):
    o_ref[...] = (jnp.dot(f_ref[...], w_ref[...],
                          preferred_element_type=jnp.float32) + b_ref[...])


_BR = RT


def _table(xn):
    br = xn.shape[0]
    return jnp.concatenate(
        [xn, jnp.ones((br, 1), jnp.float32),
         jnp.zeros((br, W128 - D - 1), jnp.float32)], axis=1)


def _norm_body(x_ref, o_ref):
    x = x_ref[...]
    n = jnp.sqrt(jnp.sum(x * x, axis=1, keepdims=True))
    o_ref[...] = _table(x / jnp.maximum(n, 1e-12))


def _layer_body(xt_ref, p_ref, c_ref, id_ref,
                cw_ref, lw_ref, lb_ref, gw_ref, gb_ref, o_ref):
    cnt = jnp.maximum(c_ref[0] + c_ref[1], 1.0)[:, 0:1]
    sagg = (p_ref[0, :, :D] + p_ref[1, :, :D]) / cnt
    h = _lrelu(jnp.dot(sagg, cw_ref[...], preferred_element_type=jnp.float32))
    x_hat = _lrelu(jnp.dot(xt_ref[:, :D], lw_ref[...],
                           preferred_element_type=jnp.float32)
                   + lb_ref[...]) + id_ref[...]
    o_ref[...] = _table(_lrelu(jnp.dot(h, gw_ref[...],
                                       preferred_element_type=jnp.float32)
                               + gb_ref[...] + x_hat))


def _tc_mlp(features, w, b):
    m = features.shape[0]
    return pl.pallas_call(
        _mlp_body,
        grid=(m // 1000,),
        in_specs=[
            pl.BlockSpec((1000, 128), lambda i: (i, 0)),
            pl.BlockSpec((128, D), lambda i: (0, 0)),
            pl.BlockSpec((1, D), lambda i: (0, 0)),
        ],
        out_specs=pl.BlockSpec((1000, D), lambda i: (i, 0)),
        out_shape=jax.ShapeDtypeStruct((m, D), jnp.float32),
    )(features, w, b.reshape(1, D))


def _tc_norm(xpad):
    return pl.pallas_call(
        _norm_body,
        grid=(R // _BR,),
        in_specs=[pl.BlockSpec((_BR, D), lambda i: (i, 0))],
        out_specs=pl.BlockSpec((_BR, W128), lambda i: (i, 0)),
        out_shape=jax.ShapeDtypeStruct((R, W128), jnp.float32),
    )(xpad)


def _tc_layer(xt, p, cnt_src, id_pad, cw, lw, lb, gw, gb):
    mat = pl.BlockSpec((D, D), lambda i: (0, 0))
    vec = pl.BlockSpec((1, D), lambda i: (0, 0))
    return pl.pallas_call(
        _layer_body,
        grid=(R // _BR,),
        in_specs=[pl.BlockSpec((_BR, W128), lambda i: (i, 0)),
                  pl.BlockSpec((NC, _BR, W128), lambda i: (0, i, 0)),
                  pl.BlockSpec((NC, _BR, 16), lambda i: (0, i, 0)),
                  pl.BlockSpec((_BR, D), lambda i: (i, 0)),
                  mat, mat, vec, mat, vec],
        out_specs=pl.BlockSpec((_BR, W128), lambda i: (i, 0)),
        out_shape=jax.ShapeDtypeStruct((R, W128), jnp.float32),
    )(xt, p, cnt_src, id_pad, cw, lw, lb.reshape(1, D), gw,
      gb.reshape(1, D))


def kernel(features, id_embedding, edge_index, preference, W_mlp, b_mlp,
           conv1_w, lin1_w, lin1_b, g1_w, g1_b,
           conv2_w, lin2_w, lin2_b, g2_w, g2_b,
           conv3_w, lin3_w, lin3_b, g3_w, g3_b):
    src = edge_index[0]
    dst = edge_index[1]
    pad = EPAD - E
    paddst = N + (jnp.arange(pad, dtype=jnp.int32) % (R - N))
    srcp = jnp.concatenate([src, jnp.zeros((pad,), jnp.int32)]
                           ).reshape(NW, CH, CS)
    dstp = jnp.concatenate([dst, paddst]).reshape(NW, CH, CS)
    padrows = jnp.zeros((R - N, D), jnp.float32)
    id_pad = jnp.concatenate([id_embedding, padrows])

    temp = _tc_mlp(features, W_mlp, b_mlp)
    xpad = jnp.concatenate([preference, temp, padrows[:R - N]], axis=0)
    xt = _tc_norm(xpad)

    cnt_src = None
    for cw, lw, lb, gw, gb in (
        (conv1_w, lin1_w, lin1_b, g1_w, g1_b),
        (conv2_w, lin2_w, lin2_b, g2_w, g2_b),
        (conv3_w, lin3_w, lin3_b, g3_w, g3_b),
    ):
        p = _sc_agg(xt, srcp, dstp)
        if cnt_src is None:
            cnt_src = p[:, :, D:D + 16]
        xt = _tc_layer(xt, p, cnt_src, id_pad, cw, lw, lb, gw, gb)
    return xt[:N, :D]


# speedup vs baseline: 1.3758x; 1.0140x over previous
import functools

import jax
import jax.numpy as jnp
from jax import lax
from jax.experimental import pallas as pl
from jax.experimental.pallas import tpu as pltpu
from jax.experimental.pallas import tpu_sc as plsc

N = 10000
D = 64
W128 = 128
E = 320000
NC = 2
NS = 16
NW = NC * NS
CS = 128
CH = 79
EPAD = NW * CH * CS
R = 10112
RT = R // NS

_mesh = plsc.VectorSubcoreMesh(core_axis_name="c", subcore_axis_name="s",
                               num_cores=NC, num_subcores=NS)


@functools.partial(
    pl.kernel,
    out_type=jax.ShapeDtypeStruct((NC, R, W128), jnp.float32),
    mesh=_mesh,
    scratch_types=[
        pltpu.VMEM((CH, CS), jnp.int32),
        pltpu.VMEM((CH, CS), jnp.int32),
        pltpu.VMEM((CS, W128), jnp.float32),
        pltpu.VMEM_SHARED((R, W128), jnp.float32),
        pltpu.SemaphoreType.DMA,
    ],
)
def _sc_agg(table_hbm, src_hbm, dst_hbm, z_hbm, out_hbm,
            src_v, dst_v, rows_v, acc, semg):
    c = lax.axis_index("c")
    s = lax.axis_index("s")
    wid = c * NS + s
    pltpu.sync_copy(z_hbm.at[pl.ds(s * RT, RT)], acc.at[pl.ds(s * RT, RT)])
    pltpu.sync_copy(src_hbm.at[wid], src_v)
    pltpu.sync_copy(dst_hbm.at[wid], dst_v)
    plsc.subcore_barrier()

    def body(j, carry):
        pltpu.async_copy(table_hbm.at[src_v.at[j]], rows_v, semg).wait()
        pltpu.sync_copy(rows_v, acc.at[dst_v.at[j]], add=True)
        return carry

    lax.fori_loop(0, CH, body, 0)
    plsc.subcore_barrier()
    pltpu.sync_copy(acc.at[pl.ds(s * RT, RT)],
                    out_hbm.at[c, pl.ds(s * RT, RT)])


def _lrelu(v):
    return jnp.where(v >= 0, v, 0.01 * v)


def _mlp_body(f_ref, w_ref, b_ref, o_ref):
    o_ref[...] = (jnp.dot(f_ref[...], w_ref[...],
                          preferred_element_type=jnp.float32) + b_ref[...])


_BR = RT


def _table(xn):
    br = xn.shape[0]
    return jnp.concatenate(
        [xn, jnp.ones((br, 1), jnp.float32),
         jnp.zeros((br, W128 - D - 1), jnp.float32)], axis=1)


def _norm_body(x_ref, o_ref):
    x = x_ref[...]
    n = jnp.sqrt(jnp.sum(x * x, axis=1, keepdims=True))
    o_ref[...] = _table(x / jnp.maximum(n, 1e-12))


def _layer_body(xt_ref, p_ref, c_ref, id_ref,
                cw_ref, lw_ref, lb_ref, gw_ref, gb_ref, o_ref):
    cnt = jnp.maximum(c_ref[0] + c_ref[1], 1.0)[:, 0:1]
    sagg = (p_ref[0, :, :D] + p_ref[1, :, :D]) / cnt
    h = _lrelu(jnp.dot(sagg, cw_ref[...], preferred_element_type=jnp.float32))
    x_hat = _lrelu(jnp.dot(xt_ref[:, :D], lw_ref[...],
                           preferred_element_type=jnp.float32)
                   + lb_ref[...]) + id_ref[...]
    o_ref[...] = _table(_lrelu(jnp.dot(h, gw_ref[...],
                                       preferred_element_type=jnp.float32)
                               + gb_ref[...] + x_hat))


def _tc_mlp(features, w, b):
    m = features.shape[0]
    return pl.pallas_call(
        _mlp_body,
        grid=(m // 1000,),
        in_specs=[
            pl.BlockSpec((1000, 128), lambda i: (i, 0)),
            pl.BlockSpec((128, D), lambda i: (0, 0)),
            pl.BlockSpec((1, D), lambda i: (0, 0)),
        ],
        out_specs=pl.BlockSpec((1000, D), lambda i: (i, 0)),
        out_shape=jax.ShapeDtypeStruct((m, D), jnp.float32),
    )(features, w, b.reshape(1, D))


def _tc_norm(xpad):
    return pl.pallas_call(
        _norm_body,
        grid=(R // _BR,),
        in_specs=[pl.BlockSpec((_BR, D), lambda i: (i, 0))],
        out_specs=pl.BlockSpec((_BR, W128), lambda i: (i, 0)),
        out_shape=jax.ShapeDtypeStruct((R, W128), jnp.float32),
    )(xpad)


def _tc_layer(xt, p, cnt_src, id_pad, cw, lw, lb, gw, gb):
    mat = pl.BlockSpec((D, D), lambda i: (0, 0))
    vec = pl.BlockSpec((1, D), lambda i: (0, 0))
    return pl.pallas_call(
        _layer_body,
        grid=(R // _BR,),
        in_specs=[pl.BlockSpec((_BR, W128), lambda i: (i, 0)),
                  pl.BlockSpec((NC, _BR, W128), lambda i: (0, i, 0)),
                  pl.BlockSpec((NC, _BR, 16), lambda i: (0, i, 0)),
                  pl.BlockSpec((_BR, D), lambda i: (i, 0)),
                  mat, mat, vec, mat, vec],
        out_specs=pl.BlockSpec((_BR, W128), lambda i: (i, 0)),
        out_shape=jax.ShapeDtypeStruct((R, W128), jnp.float32),
    )(xt, p, cnt_src, id_pad, cw, lw, lb.reshape(1, D), gw,
      gb.reshape(1, D))


def kernel(features, id_embedding, edge_index, preference, W_mlp, b_mlp,
           conv1_w, lin1_w, lin1_b, g1_w, g1_b,
           conv2_w, lin2_w, lin2_b, g2_w, g2_b,
           conv3_w, lin3_w, lin3_b, g3_w, g3_b):
    src = edge_index[0]
    dst = edge_index[1]
    pad = EPAD - E
    paddst = N + (jnp.arange(pad, dtype=jnp.int32) % (R - N))
    srcp = jnp.concatenate([src, jnp.zeros((pad,), jnp.int32)]
                           ).reshape(NW, CH, CS)
    dstp = jnp.concatenate([dst, paddst]).reshape(NW, CH, CS)
    zeros128 = jnp.zeros((R, W128), jnp.float32)
    padrows = jnp.zeros((R - N, D), jnp.float32)
    id_pad = jnp.concatenate([id_embedding, padrows])

    temp = _tc_mlp(features, W_mlp, b_mlp)
    xpad = jnp.concatenate([preference, temp, padrows[:R - N]], axis=0)
    xt = _tc_norm(xpad)

    cnt_src = None
    for cw, lw, lb, gw, gb in (
        (conv1_w, lin1_w, lin1_b, g1_w, g1_b),
        (conv2_w, lin2_w, lin2_b, g2_w, g2_b),
        (conv3_w, lin3_w, lin3_b, g3_w, g3_b),
    ):
        p = _sc_agg(xt, srcp, dstp, zeros128)
        if cnt_src is None:
            cnt_src = p[:, :, D:D + 16]
        xt = _tc_layer(xt, p, cnt_src, id_pad, cw, lw, lb, gw, gb)
    return xt[:N, :D]
